# baseline (speedup 1.0000x reference)
import functools

import jax
import jax.numpy as jnp
from jax import lax
from jax.experimental import pallas as pl
from jax.experimental.pallas import tpu as pltpu
from jax.experimental.pallas import tpu_sc as plsc

NB = 4096
S = 200
D = 64
V = 1000000
TBLK = 2048
NTB = (V + 2 * TBLK - 1) // (2 * TBLK)
VHP = NTB * TBLK
NC, NS = 2, 16
NW = NC * NS
GRPS = 128 // 16


def _make_pair_merge():
    def body(a_ref, out_ref):
        x = a_ref[...]
        out_ref[:, 0:D] = x[0:TBLK]
        out_ref[:, D:128] = x[TBLK:2 * TBLK]

    return pl.pallas_call(
        body,
        grid=(NTB,),
        in_specs=[pl.BlockSpec((2 * TBLK, D), lambda i: (i, 0))],
        out_specs=pl.BlockSpec((TBLK, 128), lambda i: (i, 0)),
        out_shape=jax.ShapeDtypeStruct((VHP, 128), jnp.float32),
    )


def _make_sc_gather():
    mesh = plsc.VectorSubcoreMesh(core_axis_name="c", subcore_axis_name="s")

    @functools.partial(
        pl.kernel,
        mesh=mesh,
        out_type=jax.ShapeDtypeStruct((S, D, NB), jnp.float32),
        scratch_types=[
            pltpu.VMEM((S, 128), jnp.int32),
            pltpu.VMEM((4, 128), jnp.int32),
            pltpu.VMEM((4, 128, 128), jnp.float32),
            pltpu.VMEM((4, D, 128), jnp.float32),
            pltpu.SemaphoreType.DMA((4,)),
            pltpu.SemaphoreType.DMA((4,)),
        ],
        compiler_params=pltpu.CompilerParams(use_tc_tiling_on_sc=True,
                                             needs_layout_passes=False),
    )
    def gather_kernel(idx_hbm, rm_hbm, out_hbm, idxv, rowsv, gbuf, obuf,
                      gsem, ssem):
        wid = lax.axis_index("s") * NC + lax.axis_index("c")
        col = wid * 128
        pltpu.sync_copy(idx_hbm.at[:, pl.ds(col, 128)], idxv)

        def prep_rows(s, p):
            for k in range(GRPS):
                ids = idxv[s, pl.ds(16 * k, 16)]
                rowsv[p, pl.ds(16 * k, 16)] = (
                    ((ids >> 12) << 11) | (ids & 2047))

        def start_gather(p):
            pltpu.async_copy(rm_hbm.at[rowsv.at[p]], gbuf.at[p], gsem.at[p])

        def wait_gather(p):
            pltpu.make_async_copy(rm_hbm.at[rowsv.at[p]], gbuf.at[p],
                                  gsem.at[p]).wait()

        def start_store(s, p):
            pltpu.async_copy(obuf.at[p], out_hbm.at[s, :, pl.ds(col, 128)],
                             ssem.at[p])

        def wait_store(s, p):
            pltpu.make_async_copy(obuf.at[p], out_hbm.at[s, :, pl.ds(col, 128)],
                                  ssem.at[p]).wait()

        def transform(s, p):
            def grp(g, carry):
                t16 = lax.iota(jnp.int32, 16) + 16 * g
                ids = idxv[s, pl.ds(16 * g, 16)]
                h64 = (ids >> 5) & D

                @plsc.parallel_loop(0, D, unroll=8)
                def floop(f):
                    vals = plsc.load_gather(gbuf.at[p], [t16, h64 + f])
                    obuf[p, f, pl.ds(16 * g, 16)] = vals

                return carry

            lax.fori_loop(0, GRPS, grp, 0)

        NBUF = 4

        for p in range(NBUF):
            prep_rows(p, p)
            start_gather(p)
        for p in range(NBUF):
            wait_gather(p)
            transform(p, p)
            start_store(p, p)
            prep_rows(p + NBUF, p)
            start_gather(p)

        def body(i, carry):
            for p in range(NBUF):
                s = NBUF * i + p
                wait_gather(p)
                wait_store(s - NBUF, p)
                transform(s, p)
                start_store(s, p)
                prep_rows(s + NBUF, p)
                start_gather(p)
            return carry

        lax.fori_loop(1, S // NBUF - 1, body, 0)

        for p in range(NBUF):
            s = S - NBUF + p
            wait_gather(p)
            wait_store(s - NBUF, p)
            transform(s, p)
            start_store(s, p)
        for p in range(NBUF):
            wait_store(S - NBUF + p, p)

    return gather_kernel


_tc_pair_merge = _make_pair_merge()
_sc_gather = _make_sc_gather()


@jax.jit
def kernel(token_ids, embedding):
    idx_t = token_ids.T.astype(jnp.int32)
    rm = _tc_pair_merge(embedding)
    out = _sc_gather(idx_t, rm)
    return out.transpose(2, 0, 1)


# speedup vs baseline: 1.0977x; 1.0977x over previous
import functools

import jax
import jax.numpy as jnp
from jax import lax
from jax.experimental import pallas as pl
from jax.experimental.pallas import tpu as pltpu
from jax.experimental.pallas import tpu_sc as plsc

NB = 4096
S = 200
D = 64
V = 1000000
TBLK = 2048
NTB = (V + 2 * TBLK - 1) // (2 * TBLK)
VHP = NTB * TBLK
NC, NS = 2, 16
NW = NC * NS
GRPS = 128 // 16


def _make_table_transpose():
    dn = (((0,), (0,)), ((), ()))

    def body(a_ref, out_ref):
        r = lax.broadcasted_iota(jnp.int32, (D, D), 0)
        c = lax.broadcasted_iota(jnp.int32, (D, D), 1)
        eye = (r == c).astype(jnp.float32)

        def tr(x):
            return lax.dot_general(x, eye, dn,
                                   precision=lax.Precision.HIGHEST,
                                   preferred_element_type=jnp.float32)

        x = a_ref[...]
        out_ref[:, 0:D] = tr(x[:, 0:TBLK])
        out_ref[:, D:128] = tr(x[:, TBLK:2 * TBLK])

    return pl.pallas_call(
        body,
        grid=(NTB,),
        in_specs=[pl.BlockSpec((D, 2 * TBLK), lambda i: (0, i))],
        out_specs=pl.BlockSpec((TBLK, 128), lambda i: (i, 0)),
        out_shape=jax.ShapeDtypeStruct((VHP, 128), jnp.float32),
    )


def _make_sc_gather():
    mesh = plsc.VectorSubcoreMesh(core_axis_name="c", subcore_axis_name="s")

    @functools.partial(
        pl.kernel,
        mesh=mesh,
        out_type=jax.ShapeDtypeStruct((S, D, NB), jnp.float32),
        scratch_types=[
            pltpu.VMEM((S, 128), jnp.int32),
            pltpu.VMEM((4, 128), jnp.int32),
            pltpu.VMEM((4, 128, 128), jnp.float32),
            pltpu.VMEM((4, D, 128), jnp.float32),
            pltpu.SemaphoreType.DMA((4,)),
            pltpu.SemaphoreType.DMA((4,)),
        ],
        compiler_params=pltpu.CompilerParams(use_tc_tiling_on_sc=True,
                                             needs_layout_passes=False),
    )
    def gather_kernel(idx_hbm, rm_hbm, out_hbm, idxv, rowsv, gbuf, obuf,
                      gsem, ssem):
        wid = lax.axis_index("s") * NC + lax.axis_index("c")
        col = wid * 128
        pltpu.sync_copy(idx_hbm.at[:, pl.ds(col, 128)], idxv)

        def prep_rows(s, p):
            for k in range(GRPS):
                ids = idxv[s, pl.ds(16 * k, 16)]
                rowsv[p, pl.ds(16 * k, 16)] = (
                    ((ids >> 12) << 11) | (ids & 2047))

        def start_gather(p):
            pltpu.async_copy(rm_hbm.at[rowsv.at[p]], gbuf.at[p], gsem.at[p])

        def wait_gather(p):
            pltpu.make_async_copy(rm_hbm.at[rowsv.at[p]], gbuf.at[p],
                                  gsem.at[p]).wait()

        def start_store(s, p):
            pltpu.async_copy(obuf.at[p], out_hbm.at[s, :, pl.ds(col, 128)],
                             ssem.at[p])

        def wait_store(s, p):
            pltpu.make_async_copy(obuf.at[p], out_hbm.at[s, :, pl.ds(col, 128)],
                                  ssem.at[p]).wait()

        def transform(s, p):
            def grp(g, carry):
                t16 = lax.iota(jnp.int32, 16) + 16 * g
                ids = idxv[s, pl.ds(16 * g, 16)]
                h64 = (ids >> 5) & D

                @plsc.parallel_loop(0, D, unroll=16)
                def floop(f):
                    vals = plsc.load_gather(gbuf.at[p], [t16, h64 + f])
                    obuf[p, f, pl.ds(16 * g, 16)] = vals

                return carry

            lax.fori_loop(0, GRPS, grp, 0)

        NBUF = 4

        for p in range(NBUF):
            prep_rows(p, p)
            start_gather(p)
        for p in range(NBUF):
            wait_gather(p)
            transform(p, p)
            start_store(p, p)
            prep_rows(p + NBUF, p)
            start_gather(p)

        def body(i, carry):
            for p in range(NBUF):
                s = NBUF * i + p
                wait_gather(p)
                wait_store(s - NBUF, p)
                transform(s, p)
                start_store(s, p)
                prep_rows(s + NBUF, p)
                start_gather(p)
            return carry

        lax.fori_loop(1, S // NBUF - 1, body, 0)

        for p in range(NBUF):
            s = S - NBUF + p
            wait_gather(p)
            wait_store(s - NBUF, p)
            transform(s, p)
            start_store(s, p)
        for p in range(NBUF):
            wait_store(S - NBUF + p, p)

    return gather_kernel


_tc_transpose = _make_table_transpose()
_sc_gather = _make_sc_gather()


@jax.jit
def kernel(token_ids, embedding):
    emb_t = embedding.T
    idx_t = token_ids.T.astype(jnp.int32)
    rm = _tc_transpose(emb_t)
    out = _sc_gather(idx_t, rm)
    return out.transpose(2, 0, 1)
